# baseline (speedup 1.0000x reference)
import functools

import jax
import jax.numpy as jnp
from jax import lax
from jax.experimental import pallas as pl
from jax.experimental.pallas import tpu as pltpu
from jax.experimental.pallas import tpu_sc as plsc

NC = 2
NS = 16
NW = NC * NS
ROW = 128
K = 8
CHUNK = K * ROW

N_NODES = 100000
NACC = 100016
ZR = NACC // NS
NPT = N_NODES // NS

EDGE_ALIGN = ROW * NW * K


def _pad_edges(x, epad, value):
    return jnp.pad(x, (0, epad - x.shape[0]), constant_values=value)


def _sc_gather(z_pad, idx):
    rows = idx.shape[1]
    rw = rows // NW
    nchunks = rw // K
    b_tot = rows * 16

    def body(z_hbm, idx_hbm, out_hbm, idx_v, rows_v, big_v, sem):
        wid = lax.axis_index("s") * NC + lax.axis_index("c")
        r0 = wid * rw
        for a in (0, 1):
            def chunk(i, _, a=a):
                row0 = r0 + i * K
                pltpu.sync_copy(idx_hbm.at[a, pl.ds(row0, K)], idx_v)
                cps = [
                    pltpu.async_copy(
                        z_hbm.at[idx_v.at[b]],
                        rows_v.at[pl.ds(b * ROW, ROW)], sem)
                    for b in range(K)
                ]
                for cp in cps:
                    cp.wait()

                def rel(j, _):
                    for k in range(8):
                        big_v[j, pl.ds(k * 16, 16)] = rows_v[j * 8 + k, :]
                    return 0
                lax.fori_loop(0, K * 16, rel, 0)
                pltpu.sync_copy(big_v, out_hbm.at[a, pl.ds(row0 * 16, K * 16)])
                return 0
            lax.fori_loop(0, nchunks, chunk, 0)

    mesh = plsc.VectorSubcoreMesh(core_axis_name="c", subcore_axis_name="s")
    return pl.kernel(
        body,
        out_type=jax.ShapeDtypeStruct((2, b_tot, 128), jnp.float32),
        mesh=mesh,
        compiler_params=pltpu.CompilerParams(use_tc_tiling_on_sc=False),
        scratch_types=[
            pltpu.VMEM((K, ROW), jnp.int32),
            pltpu.VMEM((CHUNK, 16), jnp.float32),
            pltpu.VMEM((K * 16, 128), jnp.float32),
            pltpu.SemaphoreType.DMA,
        ],
    )(z_pad, idx)


def _sc_scatter(msg128, tgt_rows, zrows, off8):
    KS = 4
    rows = tgt_rows.shape[0]
    rw = rows // NW
    nchunks = rw // KS

    def body(msg_hbm, idx_hbm, z_hbm, out_hbm, idx_v, msg_v, big_v, acc):
        c = lax.axis_index("c")
        s = lax.axis_index("s")
        wid = s * NC + c
        pltpu.sync_copy(z_hbm, acc.at[pl.ds(s * ZR, ZR)])
        plsc.subcore_barrier()

        def chunk(i, _):
            row0 = wid * rw + i * KS
            pltpu.sync_copy(idx_hbm.at[pl.ds(row0, KS)], idx_v)
            pltpu.sync_copy(
                msg_hbm.at[pl.ds(off8 + row0 * 16, KS * 16)], big_v)

            def rel(j, _):
                for k in range(8):
                    msg_v[j * 8 + k, :] = big_v[j, pl.ds(k * 16, 16)]
                return 0
            lax.fori_loop(0, KS * 16, rel, 0)
            for b in range(KS):
                pltpu.sync_copy(
                    msg_v.at[pl.ds(b * ROW, ROW)],
                    acc.at[idx_v.at[b]], add=True)
            return 0
        lax.fori_loop(0, nchunks, chunk, 0)
        plsc.subcore_barrier()
        pltpu.sync_copy(
            acc.at[pl.ds(s * NPT, NPT)],
            out_hbm.at[c, pl.ds(s * NPT, NPT)])

    mesh = plsc.VectorSubcoreMesh(core_axis_name="c", subcore_axis_name="s")
    return pl.kernel(
        body,
        out_type=jax.ShapeDtypeStruct((2, N_NODES, 16), jnp.float32),
        mesh=mesh,
        compiler_params=pltpu.CompilerParams(use_tc_tiling_on_sc=False),
        scratch_types=[
            pltpu.VMEM((KS, ROW), jnp.int32),
            pltpu.VMEM((KS * ROW, 16), jnp.float32),
            pltpu.VMEM((KS * 16, 128), jnp.float32),
            pltpu.VMEM_SHARED((NACC, 16), jnp.float32),
        ],
    )(msg128, tgt_rows, zrows)


def _edge_body(g_ref, w1_ref, b1_ref, w2_ref, b2_ref, out_ref):
    zs = g_ref[0]
    zt = g_ref[1]
    lane = lax.broadcasted_iota(jnp.int32, zs.shape, 1) % 16

    def m(lo, hi):
        return jnp.where((lane >= lo) & (lane < hi),
                         jnp.float32(1), jnp.float32(0))

    def roll(x, s):
        return pltpu.roll(x, s % 128, 1)

    d = zs - zt
    dsq = d * d
    dist0 = dsq + roll(dsq, -1) + roll(dsq, -2)
    p1 = zs * roll(zt, -1)
    m1 = zs * roll(zt, 1)
    m2 = zs * roll(zt, 2)
    p2 = zs * roll(zt, -2)
    ca = p1 - roll(m1, -1)
    cb = m2 - roll(p2, 2)
    cpc = ca * m(4, 5) + cb * m(5, 6) + roll(ca, 3) * m(6, 7)
    s = cpc * cpc
    ssum = s + roll(s, -1) + roll(s, -2)
    acp = jnp.sqrt(roll(ssum, 3) * m(7, 8))
    feats = d * m(0, 3) + roll(dist0, 3) * m(3, 4) + cpc + acp

    w1 = w1_ref[0]
    b1 = b1_ref[0]
    w2 = w2_ref[0]
    b2 = b2_ref[0]
    for p in range(4):
        sl = slice(32 * p, 32 * (p + 1))
        inp = jnp.concatenate([zs[:, sl], zt[:, sl], feats[:, sl]], axis=1)
        h = jnp.tanh(jnp.dot(inp, w1, preferred_element_type=jnp.float32)
                     + b1)
        y = jnp.dot(h, w2, preferred_element_type=jnp.float32) + b2
        out_ref[:, sl] = (y[:, 0:32]
                          * (0.5 * jnp.tanh(0.5 * y[:, 32:64]) + 0.5))


def _edge_mlp(gath128, ws, split_blocks, blk8):
    rows = gath128.shape[1]
    nblk = rows // blk8
    wmap = lambda i: (i // split_blocks, 0, 0)
    w1, b1, w2, b2 = ws
    return pl.pallas_call(
        _edge_body,
        grid=(nblk,),
        in_specs=[
            pl.BlockSpec((2, blk8, 128), lambda i: (0, i, 0)),
            pl.BlockSpec((1, 96, 128), wmap),
            pl.BlockSpec((1, 1, 128), wmap),
            pl.BlockSpec((1, 128, 64), wmap),
            pl.BlockSpec((1, 1, 64), wmap),
        ],
        out_specs=pl.BlockSpec((blk8, 128), lambda i: (i, 0)),
        out_shape=jax.ShapeDtypeStruct((rows, 128), jnp.float32),
    )(gath128, w1, b1, w2, b2)


def _node_mlp_body(z_ref, ph_ref, pw_ref, w1_ref, b1_ref, w2_ref, b2_ref,
                   out_ref):
    z = z_ref[...]
    aggh = ph_ref[0] + ph_ref[1]
    aggw = pw_ref[0] + pw_ref[1]
    inp = jnp.concatenate([z, aggh, aggw], axis=1)
    h = jnp.tanh(jnp.dot(inp, w1_ref[...], preferred_element_type=jnp.float32)
                 + b1_ref[...])
    out_ref[...] = (jnp.dot(h, w2_ref[...], preferred_element_type=jnp.float32)
                    + b2_ref[...])


def _node_mlp(z, parts_h, parts_w, w1, b1, w2, b2, blk):
    n = z.shape[0]
    nblk = n // blk
    return pl.pallas_call(
        _node_mlp_body,
        grid=(nblk,),
        in_specs=[
            pl.BlockSpec((blk, 13), lambda i: (i, 0)),
            pl.BlockSpec((2, blk, 16), lambda i: (0, i, 0)),
            pl.BlockSpec((2, blk, 16), lambda i: (0, i, 0)),
            pl.BlockSpec((45, 32), lambda i: (0, 0)),
            pl.BlockSpec((1, 32), lambda i: (0, 0)),
            pl.BlockSpec((32, 13), lambda i: (0, 0)),
            pl.BlockSpec((1, 13), lambda i: (0, 0)),
        ],
        out_specs=pl.BlockSpec((blk, 13), lambda i: (i, 0)),
        out_shape=jax.ShapeDtypeStruct((n, 13), jnp.float32),
    )(z, parts_h, parts_w, w1, b1, w2, b2)


def _bd2(w):
    z = jnp.zeros_like(w)
    return jnp.concatenate([jnp.concatenate([w, z], axis=1),
                            jnp.concatenate([z, w], axis=1)], axis=0)


def _prep_edge_weights(W1, b1, W2, b2, Wv1, bv1, Wv2, bv2):
    W1c = jnp.concatenate([W1, Wv1], axis=1)
    b1c = jnp.concatenate([b1, bv1])[None]
    A = jnp.zeros((16, 64), jnp.float32).at[0:13].set(W1c[0:13])
    Bm = jnp.zeros((16, 64), jnp.float32).at[0:13].set(W1c[13:26])
    Wf = jnp.zeros((16, 64), jnp.float32).at[0:3].set(W1c[26:29])
    Wf = Wf.at[3].set(W1c[29]).at[4:7].set(W1c[30:33]).at[7].set(W1c[33])
    W2m = jnp.zeros((64, 16), jnp.float32).at[0:32].set(W2)
    W2g = jnp.zeros((64, 16), jnp.float32).at[32:64].set(
        jnp.tile(Wv2, (1, 16)))
    b2m = jnp.tile(b2[None], (1, 2))
    b2g = jnp.tile(bv2[None], (1, 32))
    w1 = jnp.concatenate([_bd2(A), _bd2(Bm), _bd2(Wf)], axis=0)
    w2 = jnp.concatenate([_bd2(W2m), _bd2(W2g)], axis=1)
    b2cat = jnp.concatenate([b2m, b2g], axis=1)
    return (w1, jnp.tile(b1c, (1, 2)), w2, b2cat)


@jax.jit
def kernel(z_h, edge_index_h_h, edge_index_world,
           We1, be1, We2, be2, Ww1, bw1, Ww2, bw2,
           Wew1, bew1, Wew2, bew2, Www1, bww1, Www2, bww2,
           Wn1, bn1, Wn2, bn2):
    B, N, F = z_h.shape
    z = z_h[0]
    z_pad = jnp.pad(z, ((0, 0), (0, 16 - F)))

    src_hh = edge_index_h_h[0, 0].astype(jnp.int32)
    tgt_hh = edge_index_h_h[0, 1].astype(jnp.int32)
    src_w = edge_index_world[0, 0].astype(jnp.int32)
    tgt_w = edge_index_world[0, 1].astype(jnp.int32)

    e_hh, e_w = src_hh.shape[0], src_w.shape[0]
    epad_hh = -(-e_hh // EDGE_ALIGN) * EDGE_ALIGN
    epad_w = -(-e_w // EDGE_ALIGN) * EDGE_ALIGN

    idx_hh = jnp.stack([_pad_edges(src_hh, epad_hh, 0),
                        _pad_edges(tgt_hh, epad_hh, 0)]).reshape(2, -1, ROW)
    idx_w = jnp.stack([_pad_edges(src_w, epad_w, 0),
                       _pad_edges(tgt_w, epad_w, 0)]).reshape(2, -1, ROW)

    gath_hh = _sc_gather(z_pad, idx_hh)
    gath_w = _sc_gather(z_pad, idx_w)

    wsets = [_prep_edge_weights(We1, be1, We2, be2, Ww1, bw1, Ww2, bw2),
             _prep_edge_weights(Wew1, bew1, Wew2, bew2,
                                Www1, bww1, Www2, bww2)]
    ws_hh = tuple(w[None] for w in wsets[0])
    ws_w = tuple(w[None] for w in wsets[1])

    blk8 = 1024
    nblk_hh = gath_hh.shape[1] // blk8
    nblk_w = gath_w.shape[1] // blk8
    msg_hh = _edge_mlp(gath_hh, ws_hh, nblk_hh, blk8)
    msg_w = _edge_mlp(gath_w, ws_w, nblk_w, blk8)

    zrows = jnp.zeros((ZR, 16), jnp.float32)
    tgt_scat_hh = _pad_edges(tgt_hh, epad_hh, N_NODES).reshape(-1, ROW)
    tgt_scat_w = _pad_edges(tgt_w, epad_w, N_NODES).reshape(-1, ROW)
    parts_h = _sc_scatter(msg_hh, tgt_scat_hh, zrows, 0)
    parts_w = _sc_scatter(msg_w, tgt_scat_w, zrows, 0)

    delta = _node_mlp(z, parts_h, parts_w, Wn1, bn1[None], Wn2, bn2[None],
                      blk=2000)
    return delta[None]
